# baseline (speedup 1.0000x reference)
import functools

import jax
import jax.numpy as jnp
import numpy as np
from jax import lax
from jax.experimental import pallas as pl
from jax.experimental.pallas import tpu as pltpu
from jax.experimental.pallas import tpu_sc as plsc

_KS0 = 0x6D3E048F
_KS1 = 0x1022172D
_KS2 = _KS0 ^ _KS1 ^ 0x1BD11BDA

_T_HI = np.float32(0.003638321)
_T_LO = np.float32(0.003336203)
_SALT = np.float32(0.3890121)
_PEPPER = np.float32(-0.2562604)
_HI_CUT = np.float32(1.0) - _T_HI

_HI_BITS = 0xFF119000
_LO_BITS = 0x00DAA5FF

_ROTATIONS = ((13, 15, 26, 6), (17, 29, 16, 24))

_LANE = 384
_ROWS = 64 * 3 * 384


def _threefry_bits(idx):
    ks = (jnp.uint32(_KS0), jnp.uint32(_KS1), jnp.uint32(_KS2))
    x0 = jnp.full_like(idx, ks[0])
    x1 = idx + ks[1]
    for g in range(5):
        for r in _ROTATIONS[g % 2]:
            x0 = x0 + x1
            x1 = (x1 << jnp.uint32(r)) | (x1 >> jnp.uint32(32 - r))
            x1 = x0 ^ x1
        x0 = x0 + ks[(g + 1) % 3]
        x1 = x1 + ks[(g + 2) % 3] + jnp.uint32(g + 1)
    return x0 ^ x1


def _body(x_ref, o_ref, *, block_rows):
    base_row = pl.program_id(0) * block_rows
    shape = (block_rows, _LANE)
    row = jax.lax.broadcasted_iota(jnp.uint32, shape, 0)
    col = jax.lax.broadcasted_iota(jnp.uint32, shape, 1)
    idx = (jnp.uint32(base_row) + row) * jnp.uint32(_LANE) + col
    bits = _threefry_bits(idx)
    x = x_ref[...]
    out = jnp.where(bits >= jnp.uint32(_HI_BITS), jnp.float32(_SALT), x)
    out = jnp.where(bits <= jnp.uint32(_LO_BITS), jnp.float32(_PEPPER), out)
    o_ref[...] = out


_NC, _NS = 2, 16
_NW = _NC * _NS
_SC_ROWS = 20480
_TC_ROWS = _ROWS - _SC_ROWS
_SC_ROWS_W = _SC_ROWS // _NW
_CHUNK_ROWS = 64
_SC_CHUNKS = _SC_ROWS_W // _CHUNK_ROWS
_SC_PAIRS = _SC_CHUNKS // 2


def _sc_vec16(idx_base, lane, xv):
    idx = lax.convert_element_type(idx_base + lane, jnp.uint32)
    bits = _threefry_bits(idx)
    res = jnp.where(bits >= jnp.uint32(_HI_BITS), jnp.float32(_SALT), xv)
    return jnp.where(bits <= jnp.uint32(_LO_BITS), jnp.float32(_PEPPER), res)


def _sc_compute_chunk(x_v, o_v, rin, lane):
    def band(g, c1):
        @plsc.parallel_loop(0, 12)
        def _grp(j):
            t = j >> 2
            r = j & 3
            for rr in range(2):
                row_c = g * 8 + r * 2 + rr
                ib = (rin + row_c) * jnp.int32(_LANE) + t * 128
                for v in range(8):
                    col = t * 128 + v * 16
                    xv = x_v[row_c, pl.ds(col, 16)]
                    o_v[row_c, pl.ds(col, 16)] = _sc_vec16(
                        ib + v * 16, lane, xv)

        return c1

    lax.fori_loop(0, _CHUNK_ROWS // 8, band, 0)


def _sc_body(x_hbm, o_hbm, x_v0, x_v1, o_v0, o_v1, sin0, sin1, sout0, sout1):
    wid = lax.axis_index("s") * _NC + lax.axis_index("c")
    row0_in = jnp.int32(_TC_ROWS) + wid * _SC_ROWS_W
    row0_out = wid * _SC_ROWS_W
    lane = lax.broadcasted_iota(jnp.int32, (16,), 0)
    cr = _CHUNK_ROWS

    def in_start(c, buf, sem):
        pltpu.async_copy(x_hbm.at[pl.ds(row0_in + c * cr, cr)], buf, sem)

    def in_wait(c, buf, sem):
        pltpu.make_async_copy(
            x_hbm.at[pl.ds(row0_in + c * cr, cr)], buf, sem).wait()

    def out_start(c, buf, sem):
        pltpu.async_copy(buf, o_hbm.at[pl.ds(row0_out + c * cr, cr)], sem)

    def out_wait(c, buf, sem):
        pltpu.make_async_copy(
            buf, o_hbm.at[pl.ds(row0_out + c * cr, cr)], sem).wait()

    in_start(0, x_v0, sin0)

    def pair(h, carry):
        c0 = 2 * h
        c1 = c0 + 1
        in_start(c1, x_v1, sin1)
        in_wait(c0, x_v0, sin0)

        @pl.when(h > 0)
        def _():
            out_wait(c0 - 2, o_v0, sout0)

        _sc_compute_chunk(x_v0, o_v0, row0_in + c0 * cr, lane)
        out_start(c0, o_v0, sout0)

        @pl.when(h + 1 < _SC_PAIRS)
        def _():
            in_start(c0 + 2, x_v0, sin0)

        in_wait(c1, x_v1, sin1)

        @pl.when(h > 0)
        def _():
            out_wait(c1 - 2, o_v1, sout1)

        _sc_compute_chunk(x_v1, o_v1, row0_in + c1 * cr, lane)
        out_start(c1, o_v1, sout1)
        return carry

    lax.fori_loop(0, _SC_PAIRS, pair, 0)
    out_wait(_SC_CHUNKS - 2, o_v0, sout0)
    out_wait(_SC_CHUNKS - 1, o_v1, sout1)


_sc_call = functools.partial(
    pl.kernel,
    mesh=plsc.VectorSubcoreMesh(core_axis_name="c", subcore_axis_name="s"),
    out_type=jax.ShapeDtypeStruct((_SC_ROWS, _LANE), jnp.float32),
    scratch_types=[
        pltpu.VMEM((_CHUNK_ROWS, _LANE), jnp.float32),
        pltpu.VMEM((_CHUNK_ROWS, _LANE), jnp.float32),
        pltpu.VMEM((_CHUNK_ROWS, _LANE), jnp.float32),
        pltpu.VMEM((_CHUNK_ROWS, _LANE), jnp.float32),
        pltpu.SemaphoreType.DMA,
        pltpu.SemaphoreType.DMA,
        pltpu.SemaphoreType.DMA,
        pltpu.SemaphoreType.DMA,
    ],
    compiler_params=pltpu.CompilerParams(use_tc_tiling_on_sc=True),
)(_sc_body)


@jax.jit
def kernel(x):
    block_rows = 2048
    x2 = x.reshape(_ROWS, _LANE)
    out_sc = _sc_call(x2)
    out_tc = pl.pallas_call(
        functools.partial(_body, block_rows=block_rows),
        grid=(_TC_ROWS // block_rows,),
        in_specs=[pl.BlockSpec((block_rows, _LANE), lambda i: (i, 0))],
        out_specs=pl.BlockSpec((block_rows, _LANE), lambda i: (i, 0)),
        out_shape=jax.ShapeDtypeStruct((_ROWS, _LANE), jnp.float32),
        compiler_params=pltpu.CompilerParams(
            dimension_semantics=("arbitrary",),
        ),
    )(x2)
    out = lax.dynamic_update_slice(out_tc, out_sc, (_TC_ROWS, 0))
    return out.reshape(x.shape)


# speedup vs baseline: 1.0499x; 1.0499x over previous
import functools

import jax
import jax.numpy as jnp
import numpy as np
from jax import lax
from jax.experimental import pallas as pl
from jax.experimental.pallas import tpu as pltpu
from jax.experimental.pallas import tpu_sc as plsc

_KS0 = 0x6D3E048F
_KS1 = 0x1022172D
_KS2 = _KS0 ^ _KS1 ^ 0x1BD11BDA

_T_HI = np.float32(0.003638321)
_T_LO = np.float32(0.003336203)
_SALT = np.float32(0.3890121)
_PEPPER = np.float32(-0.2562604)
_HI_CUT = np.float32(1.0) - _T_HI

_HI_BITS = 0xFF119000
_LO_BITS = 0x00DAA5FF

_ROTATIONS = ((13, 15, 26, 6), (17, 29, 16, 24))

_LANE = 384
_ROWS = 64 * 3 * 384


def _threefry_bits(idx):
    ks = (jnp.uint32(_KS0), jnp.uint32(_KS1), jnp.uint32(_KS2))
    x0 = jnp.full_like(idx, ks[0])
    x1 = idx + ks[1]
    for g in range(5):
        for r in _ROTATIONS[g % 2]:
            x0 = x0 + x1
            x1 = (x1 << jnp.uint32(r)) | (x1 >> jnp.uint32(32 - r))
            x1 = x0 ^ x1
        x0 = x0 + ks[(g + 1) % 3]
        x1 = x1 + ks[(g + 2) % 3] + jnp.uint32(g + 1)
    return x0 ^ x1


def _body(x_ref, o_ref, *, block_rows):
    base_row = pl.program_id(0) * block_rows
    shape = (block_rows, _LANE)
    row = jax.lax.broadcasted_iota(jnp.uint32, shape, 0)
    col = jax.lax.broadcasted_iota(jnp.uint32, shape, 1)
    idx = (jnp.uint32(base_row) + row) * jnp.uint32(_LANE) + col
    bits = _threefry_bits(idx)
    x = x_ref[...]
    out = jnp.where(bits >= jnp.uint32(_HI_BITS), jnp.float32(_SALT), x)
    out = jnp.where(bits <= jnp.uint32(_LO_BITS), jnp.float32(_PEPPER), out)
    o_ref[...] = out


_NC, _NS = 2, 16
_NW = _NC * _NS
_SC_ROWS = 18432
_TC_ROWS = _ROWS - _SC_ROWS
_SC_ROWS_W = _SC_ROWS // _NW
_CHUNK_ROWS = 72
_SC_CHUNKS = _SC_ROWS_W // _CHUNK_ROWS
_SC_PAIRS = _SC_CHUNKS // 2


def _sc_vec16(idx_base, lane, xv):
    idx = lax.convert_element_type(idx_base + lane, jnp.uint32)
    bits = _threefry_bits(idx)
    res = jnp.where(bits >= jnp.uint32(_HI_BITS), jnp.float32(_SALT), xv)
    return jnp.where(bits <= jnp.uint32(_LO_BITS), jnp.float32(_PEPPER), res)


def _sc_compute_chunk(x_v, o_v, rin, lane):
    def band(g, c1):
        def grp(j, c2):
            t = j >> 2
            r = j & 3
            for rr in range(2):
                row_c = g * 8 + r * 2 + rr
                ib = (rin + row_c) * jnp.int32(_LANE) + t * 128
                for v in range(8):
                    col = t * 128 + v * 16
                    xv = x_v[row_c, pl.ds(col, 16)]
                    o_v[row_c, pl.ds(col, 16)] = _sc_vec16(
                        ib + v * 16, lane, xv)
            return c2

        return lax.fori_loop(0, 12, grp, c1)

    lax.fori_loop(0, _CHUNK_ROWS // 8, band, 0)


def _sc_body(x_hbm, o_hbm, x_v0, x_v1, o_v0, o_v1, sin0, sin1, sout0, sout1):
    wid = lax.axis_index("s") * _NC + lax.axis_index("c")
    row0_in = jnp.int32(_TC_ROWS) + wid * _SC_ROWS_W
    row0_out = wid * _SC_ROWS_W
    lane = lax.broadcasted_iota(jnp.int32, (16,), 0)
    cr = _CHUNK_ROWS

    def in_start(c, buf, sem):
        pltpu.async_copy(x_hbm.at[pl.ds(row0_in + c * cr, cr)], buf, sem)

    def in_wait(c, buf, sem):
        pltpu.make_async_copy(
            x_hbm.at[pl.ds(row0_in + c * cr, cr)], buf, sem).wait()

    def out_start(c, buf, sem):
        pltpu.async_copy(buf, o_hbm.at[pl.ds(row0_out + c * cr, cr)], sem)

    def out_wait(c, buf, sem):
        pltpu.make_async_copy(
            buf, o_hbm.at[pl.ds(row0_out + c * cr, cr)], sem).wait()

    in_start(0, x_v0, sin0)

    def pair(h, carry):
        c0 = 2 * h
        c1 = c0 + 1
        in_start(c1, x_v1, sin1)
        in_wait(c0, x_v0, sin0)

        @pl.when(h > 0)
        def _():
            out_wait(c0 - 2, o_v0, sout0)

        _sc_compute_chunk(x_v0, o_v0, row0_in + c0 * cr, lane)
        out_start(c0, o_v0, sout0)

        @pl.when(h + 1 < _SC_PAIRS)
        def _():
            in_start(c0 + 2, x_v0, sin0)

        in_wait(c1, x_v1, sin1)

        @pl.when(h > 0)
        def _():
            out_wait(c1 - 2, o_v1, sout1)

        _sc_compute_chunk(x_v1, o_v1, row0_in + c1 * cr, lane)
        out_start(c1, o_v1, sout1)
        return carry

    lax.fori_loop(0, _SC_PAIRS, pair, 0)
    out_wait(_SC_CHUNKS - 2, o_v0, sout0)
    out_wait(_SC_CHUNKS - 1, o_v1, sout1)


_sc_call = functools.partial(
    pl.kernel,
    mesh=plsc.VectorSubcoreMesh(core_axis_name="c", subcore_axis_name="s"),
    out_type=jax.ShapeDtypeStruct((_SC_ROWS, _LANE), jnp.float32),
    scratch_types=[
        pltpu.VMEM((_CHUNK_ROWS, _LANE), jnp.float32),
        pltpu.VMEM((_CHUNK_ROWS, _LANE), jnp.float32),
        pltpu.VMEM((_CHUNK_ROWS, _LANE), jnp.float32),
        pltpu.VMEM((_CHUNK_ROWS, _LANE), jnp.float32),
        pltpu.SemaphoreType.DMA,
        pltpu.SemaphoreType.DMA,
        pltpu.SemaphoreType.DMA,
        pltpu.SemaphoreType.DMA,
    ],
    compiler_params=pltpu.CompilerParams(use_tc_tiling_on_sc=True),
)(_sc_body)


@jax.jit
def kernel(x):
    block_rows = 2048
    x2 = x.reshape(_ROWS, _LANE)
    out_sc = _sc_call(x2)
    out_tc = pl.pallas_call(
        functools.partial(_body, block_rows=block_rows),
        grid=(_TC_ROWS // block_rows,),
        in_specs=[pl.BlockSpec((block_rows, _LANE), lambda i: (i, 0))],
        out_specs=pl.BlockSpec((block_rows, _LANE), lambda i: (i, 0)),
        out_shape=jax.ShapeDtypeStruct((_ROWS, _LANE), jnp.float32),
        compiler_params=pltpu.CompilerParams(
            dimension_semantics=("arbitrary",),
        ),
    )(x2)
    out = lax.dynamic_update_slice(out_tc, out_sc, (_TC_ROWS, 0))
    return out.reshape(x.shape)
